# Initial kernel scaffold; baseline (speedup 1.0000x reference)
import jax
import jax.numpy as jnp
from jax.experimental import pallas as pl


def kernel(x, edge_index, Wl0, Wr0, b0, Wl1, Wr1, b1, Wl2, Wr2, b2):
    raise NotImplementedError("write your pallas kernel here")


# speedup vs baseline: 3.7563x; 3.7563x over previous
import jax
import jax.numpy as jnp
from jax import lax
from jax.experimental import pallas as pl
from jax.experimental.pallas import tpu as pltpu
from jax.experimental.pallas import tpu_sc as plsc

N = 10000
NP = 10240
E = 320000
B = 128
C = 16
NC, NS = 2, 16
NW = NC * NS
RPW = 80
ER = NW * RPW
EP = ER * B
RPT = ER // NS
TRASH = 10200
STRIPE = NP // NS


def _mesh():
    return plsc.VectorSubcoreMesh(core_axis_name="c", subcore_axis_name="s",
                                  num_cores=NC, num_subcores=NS)


_PARAMS = pltpu.CompilerParams(use_tc_tiling_on_sc=False)


def _fill_rows(rows, nr, w, val):
    v16 = jnp.full((16,), val, jnp.float32)

    def frow(i, carry):
        for k in range(w // 16):
            rows[i, pl.ds(k * 16, 16)] = v16
        return carry

    lax.fori_loop(0, nr, frow, 0)


def _agg_pipeline(tab, srcr, dstr, idx_s, idx_d, rows, agg, sem, base,
                  nchunks, cnt=None):
    def chunk(cc, carry):
        pltpu.sync_copy(srcr.at[pl.ds(base + cc * C, C)], idx_s)
        pltpu.sync_copy(dstr.at[pl.ds(base + cc * C, C)], idx_d)

        def mbody(j, carry2):
            pltpu.async_copy(tab.at[idx_s.at[j]], rows, sem).wait()
            pltpu.sync_copy(rows, agg.at[idx_d.at[j]], add=True)
            if cnt is not None:
                ones, cnt_sh = cnt
                pltpu.sync_copy(ones, cnt_sh.at[idx_d.at[j]], add=True)
            return carry2

        lax.fori_loop(0, C, mbody, 0)
        return carry

    lax.fori_loop(0, nchunks, chunk, 0)


def _sc_edge_agg(table, src_i, dst_i, w, with_counts):
    out_type = [jax.ShapeDtypeStruct((NP, w), jnp.float32) for _ in range(2)]
    scratch = [
        pltpu.VMEM((C, B), jnp.int32),
        pltpu.VMEM((C, B), jnp.int32),
        pltpu.VMEM((B, w), jnp.float32),
        pltpu.VMEM_SHARED((NP, w), jnp.float32),
        pltpu.SemaphoreType.DMA,
    ]
    if with_counts:
        out_type += [jax.ShapeDtypeStruct((NP, 16), jnp.float32)
                     for _ in range(2)]
        scratch += [
            pltpu.VMEM((B, 16), jnp.float32),
            pltpu.VMEM_SHARED((NP, 16), jnp.float32),
        ]

    def body(tab, srcr, dstr, *rest):
        if with_counts:
            (p0, p1, c0, c1, idx_s, idx_d, rows, agg, sem, ones,
             cnt_sh) = rest
        else:
            p0, p1, idx_s, idx_d, rows, agg, sem = rest
        cid = lax.axis_index("c")
        sid = lax.axis_index("s")
        wid = cid * NS + sid
        base = wid * RPW
        _fill_rows(rows, B, w, 0.0)
        for k in range(STRIPE // B):
            pltpu.sync_copy(rows, agg.at[pl.ds(sid * STRIPE + k * B, B)])
        if with_counts:
            _fill_rows(ones, B, 16, 0.0)
            for k in range(STRIPE // B):
                pltpu.sync_copy(ones.at[pl.ds(0, B)],
                                cnt_sh.at[pl.ds(sid * STRIPE + k * B, B)])
            _fill_rows(ones, B, 16, 1.0)
        plsc.subcore_barrier()
        cnt = (ones, cnt_sh) if with_counts else None
        _agg_pipeline(tab, srcr, dstr, idx_s, idx_d, rows, agg, sem, base,
                      RPW // C, cnt)
        plsc.subcore_barrier()

        @pl.when(cid == 0)
        def _():
            pltpu.sync_copy(agg.at[pl.ds(sid * STRIPE, STRIPE)],
                            p0.at[pl.ds(sid * STRIPE, STRIPE)])
            if with_counts:
                pltpu.sync_copy(cnt_sh.at[pl.ds(sid * STRIPE, STRIPE)],
                                c0.at[pl.ds(sid * STRIPE, STRIPE)])

        @pl.when(cid == 1)
        def _():
            pltpu.sync_copy(agg.at[pl.ds(sid * STRIPE, STRIPE)],
                            p1.at[pl.ds(sid * STRIPE, STRIPE)])
            if with_counts:
                pltpu.sync_copy(cnt_sh.at[pl.ds(sid * STRIPE, STRIPE)],
                                c1.at[pl.ds(sid * STRIPE, STRIPE)])

    fn = pl.kernel(body, out_type=out_type, mesh=_mesh(),
                   scratch_types=scratch, compiler_params=_PARAMS)
    return fn(table, src_i, dst_i)


def _sc_feat_agg(h0a, h0b, src_i, dst_i):
    w = 128
    out_type = [jax.ShapeDtypeStruct((NP, w), jnp.float32) for _ in range(2)]
    scratch = [
        pltpu.VMEM((C, B), jnp.int32),
        pltpu.VMEM((C, B), jnp.int32),
        pltpu.VMEM((B, w), jnp.float32),
        pltpu.VMEM_SHARED((NP, w), jnp.float32),
        pltpu.SemaphoreType.DMA,
    ]

    def body(taba, tabb, srcr, dstr, q0, q1, idx_s, idx_d, rows, agg, sem):
        cid = lax.axis_index("c")
        sid = lax.axis_index("s")
        base = sid * RPT
        _fill_rows(rows, B, w, 0.0)
        for k in range(STRIPE // B):
            pltpu.sync_copy(rows, agg.at[pl.ds(sid * STRIPE + k * B, B)])
        plsc.subcore_barrier()

        @pl.when(cid == 0)
        def _():
            _agg_pipeline(taba, srcr, dstr, idx_s, idx_d, rows, agg, sem,
                          base, RPT // C)

        @pl.when(cid == 1)
        def _():
            _agg_pipeline(tabb, srcr, dstr, idx_s, idx_d, rows, agg, sem,
                          base, RPT // C)

        plsc.subcore_barrier()

        @pl.when(cid == 0)
        def _():
            pltpu.sync_copy(agg.at[pl.ds(sid * STRIPE, STRIPE)],
                            q0.at[pl.ds(sid * STRIPE, STRIPE)])

        @pl.when(cid == 1)
        def _():
            pltpu.sync_copy(agg.at[pl.ds(sid * STRIPE, STRIPE)],
                            q1.at[pl.ds(sid * STRIPE, STRIPE)])

    fn = pl.kernel(body, out_type=out_type, mesh=_mesh(),
                   scratch_types=scratch, compiler_params=_PARAMS)
    return fn(h0a, h0b, src_i, dst_i)


TM = 256
GRID = NP // TM


def _tc_layer0(p0, p1, c0, c1, x, wl, wr, b):
    def body(p0r, p1r, c0r, c1r, xr, wlr, wrr, br, oa, ob, oi):
        cnt = c0r[...][:, :1] + c1r[...][:, :1]
        inv = 1.0 / jnp.maximum(cnt, 1.0)
        mean = (p0r[...] + p1r[...]) * inv
        h = (jnp.dot(mean, wlr[...], preferred_element_type=jnp.float32)
             + jnp.dot(xr[...], wrr[...], preferred_element_type=jnp.float32)
             + br[...])
        h = jnp.maximum(h, 0.0)
        oa[...] = h[:, :128]
        ob[...] = h[:, 128:]
        oi[...] = inv

    return pl.pallas_call(
        body,
        grid=(GRID,),
        in_specs=[
            pl.BlockSpec((TM, 128), lambda i: (i, 0)),
            pl.BlockSpec((TM, 128), lambda i: (i, 0)),
            pl.BlockSpec((TM, 16), lambda i: (i, 0)),
            pl.BlockSpec((TM, 16), lambda i: (i, 0)),
            pl.BlockSpec((TM, 128), lambda i: (i, 0)),
            pl.BlockSpec((128, 256), lambda i: (0, 0)),
            pl.BlockSpec((128, 256), lambda i: (0, 0)),
            pl.BlockSpec((1, 256), lambda i: (0, 0)),
        ],
        out_specs=[pl.BlockSpec((TM, 128), lambda i: (i, 0)),
                   pl.BlockSpec((TM, 128), lambda i: (i, 0)),
                   pl.BlockSpec((TM, 1), lambda i: (i, 0))],
        out_shape=[jax.ShapeDtypeStruct((NP, 128), jnp.float32),
                   jax.ShapeDtypeStruct((NP, 128), jnp.float32),
                   jax.ShapeDtypeStruct((NP, 1), jnp.float32)],
    )(p0, p1, c0, c1, x, wl, wr, b)


def _tc_mid(q0, q1, inv, h0a, h0b, wl1, wr1, b1, wl2, wr2, b2):
    def body(q0r, q1r, ivr, har, hbr, wl1r, wr1r, b1r, wl2r, wr2r, b2r,
             oy, oz):
        inv_ = ivr[...]
        mean = jnp.concatenate([q0r[...] * inv_, q1r[...] * inv_], axis=1)
        h0 = jnp.concatenate([har[...], hbr[...]], axis=1)
        h1 = (jnp.dot(mean, wl1r[...], preferred_element_type=jnp.float32)
              + jnp.dot(h0, wr1r[...], preferred_element_type=jnp.float32)
              + b1r[...])
        h1 = jnp.maximum(h1, 0.0)
        oy[...] = jnp.dot(h1, wl2r[...], preferred_element_type=jnp.float32)
        oz[...] = (jnp.dot(h1, wr2r[...], preferred_element_type=jnp.float32)
                   + b2r[...])

    return pl.pallas_call(
        body,
        grid=(GRID,),
        in_specs=[
            pl.BlockSpec((TM, 128), lambda i: (i, 0)),
            pl.BlockSpec((TM, 128), lambda i: (i, 0)),
            pl.BlockSpec((TM, 1), lambda i: (i, 0)),
            pl.BlockSpec((TM, 128), lambda i: (i, 0)),
            pl.BlockSpec((TM, 128), lambda i: (i, 0)),
            pl.BlockSpec((256, 256), lambda i: (0, 0)),
            pl.BlockSpec((256, 256), lambda i: (0, 0)),
            pl.BlockSpec((1, 256), lambda i: (0, 0)),
            pl.BlockSpec((256, 64), lambda i: (0, 0)),
            pl.BlockSpec((256, 64), lambda i: (0, 0)),
            pl.BlockSpec((1, 64), lambda i: (0, 0)),
        ],
        out_specs=[pl.BlockSpec((TM, 64), lambda i: (i, 0))] * 2,
        out_shape=[jax.ShapeDtypeStruct((NP, 64), jnp.float32)] * 2,
    )(q0, q1, inv, h0a, h0b, wl1, wr1, b1, wl2, wr2, b2)


def _tc_out(r0, r1, inv, z2):
    def body(r0r, r1r, ivr, zr, o):
        pre = (r0r[...] + r1r[...]) * ivr[...] + zr[...]
        col = lax.broadcasted_iota(jnp.int32, (TM, 64), 1)
        valid = col < 40
        prem = jnp.where(valid, pre, -jnp.inf)
        m = jnp.max(prem, axis=1, keepdims=True)
        e = jnp.where(valid, jnp.exp(prem - m), 0.0)
        s = jnp.sum(e, axis=1, keepdims=True)
        o[...] = prem - m - jnp.log(s)

    return pl.pallas_call(
        body,
        grid=(GRID,),
        in_specs=[
            pl.BlockSpec((TM, 64), lambda i: (i, 0)),
            pl.BlockSpec((TM, 64), lambda i: (i, 0)),
            pl.BlockSpec((TM, 1), lambda i: (i, 0)),
            pl.BlockSpec((TM, 64), lambda i: (i, 0)),
        ],
        out_specs=pl.BlockSpec((TM, 64), lambda i: (i, 0)),
        out_shape=jax.ShapeDtypeStruct((NP, 64), jnp.float32),
    )(r0, r1, inv, z2)


def kernel(x, edge_index, Wl0, Wr0, b0, Wl1, Wr1, b1, Wl2, Wr2, b2):
    src = edge_index[0].astype(jnp.int32)
    dst = edge_index[1].astype(jnp.int32)
    pad = EP - E
    src_i = jnp.concatenate([src, jnp.zeros((pad,), jnp.int32)]).reshape(ER, B)
    dst_i = jnp.concatenate([dst, jnp.full((pad,), TRASH, jnp.int32)]
                            ).reshape(ER, B)
    xp = jnp.pad(x, ((0, NP - N), (0, 0)))
    wl2p = jnp.pad(Wl2, ((0, 0), (0, 24)))
    wr2p = jnp.pad(Wr2, ((0, 0), (0, 24)))
    b2p = jnp.pad(b2, (0, 24)).reshape(1, 64)
    b0r = b0.reshape(1, -1)
    b1r = b1.reshape(1, -1)

    p0, p1, c0, c1 = _sc_edge_agg(xp, src_i, dst_i, 128, True)
    h0a, h0b, inv = _tc_layer0(p0, p1, c0, c1, xp, Wl0, Wr0, b0r)
    q0, q1 = _sc_feat_agg(h0a, h0b, src_i, dst_i)
    y2, z2 = _tc_mid(q0, q1, inv, h0a, h0b, Wl1, Wr1, b1r, wl2p, wr2p, b2p)
    r0, r1 = _sc_edge_agg(y2, src_i, dst_i, 64, False)
    outp = _tc_out(r0, r1, inv, z2)
    return outp[:N, :40]


# speedup vs baseline: 9.8999x; 2.6355x over previous
import jax
import jax.numpy as jnp
from jax import lax
from jax.experimental import pallas as pl
from jax.experimental.pallas import tpu as pltpu
from jax.experimental.pallas import tpu_sc as plsc

N = 10000
NP = 10240
E = 320000
B = 128
C = 10
NC, NS = 2, 16
NW = NC * NS
RPW = 80
ER = NW * RPW
EP = ER * B
RPT = ER // NS
TRASH = 10200
STRIPE = NP // NS


def _mesh():
    return plsc.VectorSubcoreMesh(core_axis_name="c", subcore_axis_name="s",
                                  num_cores=NC, num_subcores=NS)


_PARAMS = pltpu.CompilerParams(use_tc_tiling_on_sc=False)


def _fill_rows(rows, nr, w, val):
    v16 = jnp.full((16,), val, jnp.float32)

    def frow(i, carry):
        for k in range(w // 16):
            rows[i, pl.ds(k * 16, 16)] = v16
        return carry

    lax.fori_loop(0, nr, frow, 0)


def _fill3(ref3, w, val):
    v16 = jnp.full((16,), val, jnp.float32)

    def frow(i, carry):
        for s in range(2):
            for k in range(w // 16):
                ref3[s, i, pl.ds(k * 16, 16)] = v16
        return carry

    lax.fori_loop(0, B, frow, 0)


def _agg_pipeline(tab, srcr, dstr, idx_s, idx_d, rows, agg, sem, base,
                  nchunks, cnt=None):
    def chunk(cc, carry):
        pltpu.sync_copy(srcr.at[pl.ds(base + cc * C, C)], idx_s)
        pltpu.sync_copy(dstr.at[pl.ds(base + cc * C, C)], idx_d)
        pltpu.async_copy(tab.at[idx_s.at[0]], rows.at[0], sem)

        def mbody(j, carry2):
            s = lax.rem(j, 2)
            ns = lax.rem(j + 1, 2)
            pltpu.make_async_copy(tab.at[idx_s.at[j]], rows.at[s],
                                  sem).wait()

            @pl.when(j < C - 1)
            def _():
                pltpu.async_copy(tab.at[idx_s.at[j + 1]], rows.at[ns], sem)

            pltpu.sync_copy(rows.at[s], agg.at[idx_d.at[j]], add=True)
            if cnt is not None:
                ones, cnt_sh = cnt
                pltpu.sync_copy(ones, cnt_sh.at[idx_d.at[j]], add=True)
            return carry2

        lax.fori_loop(0, C, mbody, 0)
        return carry

    lax.fori_loop(0, nchunks, chunk, 0)


def _sc_edge_agg(table, src_i, dst_i, w, with_counts):
    out_type = [jax.ShapeDtypeStruct((NP, w), jnp.float32) for _ in range(2)]
    scratch = [
        pltpu.VMEM((C, B), jnp.int32),
        pltpu.VMEM((C, B), jnp.int32),
        pltpu.VMEM((2, B, w), jnp.float32),
        pltpu.VMEM_SHARED((NP, w), jnp.float32),
        pltpu.SemaphoreType.DMA,
    ]
    if with_counts:
        out_type += [jax.ShapeDtypeStruct((NP, 16), jnp.float32)
                     for _ in range(2)]
        scratch += [
            pltpu.VMEM((B, 16), jnp.float32),
            pltpu.VMEM_SHARED((NP, 16), jnp.float32),
        ]

    def body(tab, srcr, dstr, *rest):
        if with_counts:
            (p0, p1, c0, c1, idx_s, idx_d, rows, agg, sem, ones,
             cnt_sh) = rest
        else:
            p0, p1, idx_s, idx_d, rows, agg, sem = rest
        cid = lax.axis_index("c")
        sid = lax.axis_index("s")
        wid = cid * NS + sid
        base = wid * RPW
        _fill3(rows, w, 0.0)
        for k in range(STRIPE // B):
            pltpu.sync_copy(rows.at[0],
                            agg.at[pl.ds(sid * STRIPE + k * B, B)])
        if with_counts:
            _fill_rows(ones, B, 16, 0.0)
            for k in range(STRIPE // B):
                pltpu.sync_copy(ones.at[pl.ds(0, B)],
                                cnt_sh.at[pl.ds(sid * STRIPE + k * B, B)])
            _fill_rows(ones, B, 16, 1.0)
        plsc.subcore_barrier()
        cnt = (ones, cnt_sh) if with_counts else None
        _agg_pipeline(tab, srcr, dstr, idx_s, idx_d, rows, agg, sem, base,
                      RPW // C, cnt)
        plsc.subcore_barrier()

        @pl.when(cid == 0)
        def _():
            pltpu.sync_copy(agg.at[pl.ds(sid * STRIPE, STRIPE)],
                            p0.at[pl.ds(sid * STRIPE, STRIPE)])
            if with_counts:
                pltpu.sync_copy(cnt_sh.at[pl.ds(sid * STRIPE, STRIPE)],
                                c0.at[pl.ds(sid * STRIPE, STRIPE)])

        @pl.when(cid == 1)
        def _():
            pltpu.sync_copy(agg.at[pl.ds(sid * STRIPE, STRIPE)],
                            p1.at[pl.ds(sid * STRIPE, STRIPE)])
            if with_counts:
                pltpu.sync_copy(cnt_sh.at[pl.ds(sid * STRIPE, STRIPE)],
                                c1.at[pl.ds(sid * STRIPE, STRIPE)])

    fn = pl.kernel(body, out_type=out_type, mesh=_mesh(),
                   scratch_types=scratch, compiler_params=_PARAMS)
    return fn(table, src_i, dst_i)


def _sc_feat_agg(h0a, h0b, src_i, dst_i):
    w = 128
    out_type = [jax.ShapeDtypeStruct((NP, w), jnp.float32) for _ in range(2)]
    scratch = [
        pltpu.VMEM((C, B), jnp.int32),
        pltpu.VMEM((C, B), jnp.int32),
        pltpu.VMEM((2, B, w), jnp.float32),
        pltpu.VMEM_SHARED((NP, w), jnp.float32),
        pltpu.SemaphoreType.DMA,
    ]

    def body(taba, tabb, srcr, dstr, q0, q1, idx_s, idx_d, rows, agg, sem):
        cid = lax.axis_index("c")
        sid = lax.axis_index("s")
        base = sid * RPT
        _fill3(rows, w, 0.0)
        for k in range(STRIPE // B):
            pltpu.sync_copy(rows.at[0],
                            agg.at[pl.ds(sid * STRIPE + k * B, B)])
        plsc.subcore_barrier()

        @pl.when(cid == 0)
        def _():
            _agg_pipeline(taba, srcr, dstr, idx_s, idx_d, rows, agg, sem,
                          base, RPT // C)

        @pl.when(cid == 1)
        def _():
            _agg_pipeline(tabb, srcr, dstr, idx_s, idx_d, rows, agg, sem,
                          base, RPT // C)

        plsc.subcore_barrier()

        @pl.when(cid == 0)
        def _():
            pltpu.sync_copy(agg.at[pl.ds(sid * STRIPE, STRIPE)],
                            q0.at[pl.ds(sid * STRIPE, STRIPE)])

        @pl.when(cid == 1)
        def _():
            pltpu.sync_copy(agg.at[pl.ds(sid * STRIPE, STRIPE)],
                            q1.at[pl.ds(sid * STRIPE, STRIPE)])

    fn = pl.kernel(body, out_type=out_type, mesh=_mesh(),
                   scratch_types=scratch, compiler_params=_PARAMS)
    return fn(h0a, h0b, src_i, dst_i)


TM = 256
GRID = NP // TM


def _tc_layer0(p0, p1, c0, c1, x, wl, wr, b):
    def body(p0r, p1r, c0r, c1r, xr, wlr, wrr, br, oa, ob, oi):
        cnt = c0r[...][:, :1] + c1r[...][:, :1]
        inv = 1.0 / jnp.maximum(cnt, 1.0)
        mean = (p0r[...] + p1r[...]) * inv
        h = (jnp.dot(mean, wlr[...], preferred_element_type=jnp.float32)
             + jnp.dot(xr[...], wrr[...], preferred_element_type=jnp.float32)
             + br[...])
        h = jnp.maximum(h, 0.0)
        oa[...] = h[:, :128]
        ob[...] = h[:, 128:]
        oi[...] = inv

    return pl.pallas_call(
        body,
        grid=(GRID,),
        in_specs=[
            pl.BlockSpec((TM, 128), lambda i: (i, 0)),
            pl.BlockSpec((TM, 128), lambda i: (i, 0)),
            pl.BlockSpec((TM, 16), lambda i: (i, 0)),
            pl.BlockSpec((TM, 16), lambda i: (i, 0)),
            pl.BlockSpec((TM, 128), lambda i: (i, 0)),
            pl.BlockSpec((128, 256), lambda i: (0, 0)),
            pl.BlockSpec((128, 256), lambda i: (0, 0)),
            pl.BlockSpec((1, 256), lambda i: (0, 0)),
        ],
        out_specs=[pl.BlockSpec((TM, 128), lambda i: (i, 0)),
                   pl.BlockSpec((TM, 128), lambda i: (i, 0)),
                   pl.BlockSpec((TM, 1), lambda i: (i, 0))],
        out_shape=[jax.ShapeDtypeStruct((NP, 128), jnp.float32),
                   jax.ShapeDtypeStruct((NP, 128), jnp.float32),
                   jax.ShapeDtypeStruct((NP, 1), jnp.float32)],
    )(p0, p1, c0, c1, x, wl, wr, b)


def _tc_mid(q0, q1, inv, h0a, h0b, wl1, wr1, b1, wl2, wr2, b2):
    def body(q0r, q1r, ivr, har, hbr, wl1r, wr1r, b1r, wl2r, wr2r, b2r,
             oy, oz):
        inv_ = ivr[...]
        mean = jnp.concatenate([q0r[...] * inv_, q1r[...] * inv_], axis=1)
        h0 = jnp.concatenate([har[...], hbr[...]], axis=1)
        h1 = (jnp.dot(mean, wl1r[...], preferred_element_type=jnp.float32)
              + jnp.dot(h0, wr1r[...], preferred_element_type=jnp.float32)
              + b1r[...])
        h1 = jnp.maximum(h1, 0.0)
        oy[...] = jnp.dot(h1, wl2r[...], preferred_element_type=jnp.float32)
        oz[...] = (jnp.dot(h1, wr2r[...], preferred_element_type=jnp.float32)
                   + b2r[...])

    return pl.pallas_call(
        body,
        grid=(GRID,),
        in_specs=[
            pl.BlockSpec((TM, 128), lambda i: (i, 0)),
            pl.BlockSpec((TM, 128), lambda i: (i, 0)),
            pl.BlockSpec((TM, 1), lambda i: (i, 0)),
            pl.BlockSpec((TM, 128), lambda i: (i, 0)),
            pl.BlockSpec((TM, 128), lambda i: (i, 0)),
            pl.BlockSpec((256, 256), lambda i: (0, 0)),
            pl.BlockSpec((256, 256), lambda i: (0, 0)),
            pl.BlockSpec((1, 256), lambda i: (0, 0)),
            pl.BlockSpec((256, 64), lambda i: (0, 0)),
            pl.BlockSpec((256, 64), lambda i: (0, 0)),
            pl.BlockSpec((1, 64), lambda i: (0, 0)),
        ],
        out_specs=[pl.BlockSpec((TM, 64), lambda i: (i, 0))] * 2,
        out_shape=[jax.ShapeDtypeStruct((NP, 64), jnp.float32)] * 2,
    )(q0, q1, inv, h0a, h0b, wl1, wr1, b1, wl2, wr2, b2)


def _tc_out(r0, r1, inv, z2):
    def body(r0r, r1r, ivr, zr, o):
        pre = (r0r[...] + r1r[...]) * ivr[...] + zr[...]
        col = lax.broadcasted_iota(jnp.int32, (TM, 64), 1)
        valid = col < 40
        prem = jnp.where(valid, pre, -jnp.inf)
        m = jnp.max(prem, axis=1, keepdims=True)
        e = jnp.where(valid, jnp.exp(prem - m), 0.0)
        s = jnp.sum(e, axis=1, keepdims=True)
        o[...] = prem - m - jnp.log(s)

    return pl.pallas_call(
        body,
        grid=(GRID,),
        in_specs=[
            pl.BlockSpec((TM, 64), lambda i: (i, 0)),
            pl.BlockSpec((TM, 64), lambda i: (i, 0)),
            pl.BlockSpec((TM, 1), lambda i: (i, 0)),
            pl.BlockSpec((TM, 64), lambda i: (i, 0)),
        ],
        out_specs=pl.BlockSpec((TM, 64), lambda i: (i, 0)),
        out_shape=jax.ShapeDtypeStruct((NP, 64), jnp.float32),
    )(r0, r1, inv, z2)


def kernel(x, edge_index, Wl0, Wr0, b0, Wl1, Wr1, b1, Wl2, Wr2, b2):
    src = edge_index[0].astype(jnp.int32)
    dst = edge_index[1].astype(jnp.int32)
    pad = EP - E
    pad_ids = lax.iota(jnp.int32, pad)
    src_i = jnp.concatenate([src, pad_ids % N]).reshape(ER, B)
    dst_i = jnp.concatenate([dst, N + pad_ids % (NP - N)]).reshape(ER, B)
    xp = jnp.pad(x, ((0, NP - N), (0, 0)))
    wl2p = jnp.pad(Wl2, ((0, 0), (0, 24)))
    wr2p = jnp.pad(Wr2, ((0, 0), (0, 24)))
    b2p = jnp.pad(b2, (0, 24)).reshape(1, 64)
    b0r = b0.reshape(1, -1)
    b1r = b1.reshape(1, -1)

    p0, p1, c0, c1 = _sc_edge_agg(xp, src_i, dst_i, 128, True)
    h0a, h0b, inv = _tc_layer0(p0, p1, c0, c1, xp, Wl0, Wr0, b0r)
    q0, q1 = _sc_feat_agg(h0a, h0b, src_i, dst_i)
    y2, z2 = _tc_mid(q0, q1, inv, h0a, h0b, Wl1, Wr1, b1r, wl2p, wr2p, b2p)
    r0, r1 = _sc_edge_agg(y2, src_i, dst_i, 64, False)
    outp = _tc_out(r0, r1, inv, z2)
    return outp[:N, :40]
